# baseline (speedup 1.0000x reference)
import functools

import jax
import jax.numpy as jnp
from jax import lax
from jax.experimental import pallas as pl
from jax.experimental.pallas import tpu as pltpu
from jax.experimental.pallas import tpu_sc as plsc

D = 128
E = 160000
E2 = 2 * E
L = 16
NC, NS = 2, 16
NW = NC * NS
C = 128
NCHUNKS = E2 // C
BASE_PER_W = NCHUNKS // NW
EXTRA = NCHUNKS - BASE_PER_W * NW

_PERM_DNUMS = lax.GatherDimensionNumbers(
    offset_dims=(), collapsed_slice_dims=(0,), start_index_map=(0,))


def _permute(x, idx):
    return lax.gather(x, idx[:, None], _PERM_DNUMS, slice_sizes=(1,),
                      mode=lax.GatherScatterMode.PROMISE_IN_BOUNDS)


@functools.partial(
    pl.kernel,
    out_type=jax.ShapeDtypeStruct((E2,), jnp.float32),
    mesh=plsc.VectorSubcoreMesh(core_axis_name="c", subcore_axis_name="s"),
    scratch_types=[
        pltpu.VMEM((C,), jnp.int32), pltpu.VMEM((C,), jnp.int32),
        pltpu.VMEM((C,), jnp.int32), pltpu.VMEM((C,), jnp.int32),
        pltpu.VMEM((C, D), jnp.float32), pltpu.VMEM((C, D), jnp.float32),
        pltpu.VMEM((C, D), jnp.float32), pltpu.VMEM((C, D), jnp.float32),
        pltpu.VMEM((C,), jnp.float32), pltpu.VMEM((C,), jnp.float32),
        pltpu.SemaphoreType.DMA, pltpu.SemaphoreType.DMA,
        pltpu.SemaphoreType.DMA, pltpu.SemaphoreType.DMA,
        pltpu.SemaphoreType.DMA, pltpu.SemaphoreType.DMA,
    ],
)
def _edge_scores(user_hbm, item_hbm, ue_hbm, ie_hbm, out_hbm,
                 uidx0, uidx1, iidx0, iidx1,
                 urows0, urows1, vrows0, vrows1,
                 outv0, outv1,
                 semi0, semi1, semg0, semg1, semo0, semo1):
    wid = lax.axis_index("s") * NC + lax.axis_index("c")
    lane = lax.iota(jnp.int32, L)
    perms = [jnp.bitwise_xor(lane, s) for s in (8, 4, 2, 1)]

    uidx = (uidx0, uidx1)
    iidx = (iidx0, iidx1)
    urows = (urows0, urows1)
    vrows = (vrows0, vrows1)
    outv = (outv0, outv1)
    semi = (semi0, semi1)
    semg = (semg0, semg1)
    semo = (semo0, semo1)

    def base_of(k):
        return (wid + k * NW) * C

    def start_idx(k, b):
        pltpu.async_copy(ue_hbm.at[pl.ds(base_of(k), C)], uidx[b], semi[b])
        pltpu.async_copy(ie_hbm.at[pl.ds(base_of(k), C)], iidx[b], semi[b])

    def wait_idx(b):
        pltpu.make_async_copy(ue_hbm.at[pl.ds(0, C)], uidx[b], semi[b]).wait()
        pltpu.make_async_copy(ie_hbm.at[pl.ds(0, C)], iidx[b], semi[b]).wait()

    def start_gather(b):
        pltpu.async_copy(user_hbm.at[uidx[b]], urows[b], semg[b])
        pltpu.async_copy(item_hbm.at[iidx[b]], vrows[b], semg[b])

    def wait_gather(b):
        pltpu.make_async_copy(user_hbm.at[uidx[b]], urows[b], semg[b]).wait()
        pltpu.make_async_copy(item_hbm.at[iidx[b]], vrows[b], semg[b]).wait()

    def compute(b):
        def grp_body(g, _):
            out_vec = jnp.zeros((L,), jnp.float32)
            for e in range(L):
                row = g * L + e
                acc = urows[b][row, pl.ds(0, L)] * vrows[b][row, pl.ds(0, L)]
                for kk in range(1, D // L):
                    a = urows[b][row, pl.ds(kk * L, L)]
                    v = vrows[b][row, pl.ds(kk * L, L)]
                    acc = acc + a * v
                for p in perms:
                    acc = acc + _permute(acc, p)
                out_vec = jnp.where(lane == e, acc, out_vec)
            outv[b][pl.ds(g * L, L)] = out_vec
            return 0

        lax.fori_loop(0, C // L, grp_body, 0)

    def start_out(k, b):
        pltpu.async_copy(outv[b], out_hbm.at[pl.ds(base_of(k), C)], semo[b])

    def wait_out(b):
        pltpu.make_async_copy(outv[b], out_hbm.at[pl.ds(0, C)],
                              semo[b]).wait()

    def loop_body(k, _):
        pltpu.sync_copy(ue_hbm.at[pl.ds(base_of(k), C)], uidx[0])
        pltpu.sync_copy(ie_hbm.at[pl.ds(base_of(k), C)], iidx[0])
        start_gather(0)
        wait_gather(0)
        compute(0)
        pltpu.sync_copy(outv[0], out_hbm.at[pl.ds(base_of(k), C)])
        return 0

    nchunks_w = jnp.where(wid < EXTRA, BASE_PER_W + 1, BASE_PER_W)
    lax.fori_loop(0, nchunks_w, loop_body, 0)


def kernel(user_embedding, item_embedding, pos_edges, neg_edges):
    ue = jnp.concatenate([pos_edges[0], neg_edges[0]])
    ie = jnp.concatenate([pos_edges[1], neg_edges[1]])
    out = _edge_scores(user_embedding, item_embedding, ue, ie)
    return (out[:E, None], out[E:E2, None])


# speedup vs baseline: 1.7786x; 1.7786x over previous
import functools

import jax
import jax.numpy as jnp
from jax import lax
from jax.experimental import pallas as pl
from jax.experimental.pallas import tpu as pltpu
from jax.experimental.pallas import tpu_sc as plsc

D = 128
E = 160000
E2 = 2 * E
L = 16
NC, NS = 2, 16
NW = NC * NS
C = 128
NCHUNKS = E2 // C
K = -(-NCHUNKS // NW)
KPAIR = (K + 1) // 2

_PERM_DNUMS = lax.GatherDimensionNumbers(
    offset_dims=(), collapsed_slice_dims=(0,), start_index_map=(0,))


def _permute(x, idx):
    return lax.gather(x, idx[:, None], _PERM_DNUMS, slice_sizes=(1,),
                      mode=lax.GatherScatterMode.PROMISE_IN_BOUNDS)


@functools.partial(
    pl.kernel,
    out_type=jax.ShapeDtypeStruct((E2,), jnp.float32),
    mesh=plsc.VectorSubcoreMesh(core_axis_name="c", subcore_axis_name="s"),
    scratch_types=[
        pltpu.VMEM((C,), jnp.int32), pltpu.VMEM((C,), jnp.int32),
        pltpu.VMEM((C,), jnp.int32), pltpu.VMEM((C,), jnp.int32),
        pltpu.VMEM((C, D), jnp.float32), pltpu.VMEM((C, D), jnp.float32),
        pltpu.VMEM((C, D), jnp.float32), pltpu.VMEM((C, D), jnp.float32),
        pltpu.VMEM((C,), jnp.float32), pltpu.VMEM((C,), jnp.float32),
        pltpu.SemaphoreType.DMA, pltpu.SemaphoreType.DMA,
        pltpu.SemaphoreType.DMA, pltpu.SemaphoreType.DMA,
        pltpu.SemaphoreType.DMA, pltpu.SemaphoreType.DMA,
    ],
)
def _edge_scores(user_hbm, item_hbm, ue_hbm, ie_hbm, out_hbm,
                 uidx0, uidx1, iidx0, iidx1,
                 urows0, urows1, vrows0, vrows1,
                 outv0, outv1,
                 semi0, semi1, semg0, semg1, semo0, semo1):
    wid = lax.axis_index("s") * NC + lax.axis_index("c")
    lane = lax.iota(jnp.int32, L)
    perms = [jnp.bitwise_xor(lane, s) for s in (8, 4, 2, 1)]

    uidx = (uidx0, uidx1)
    iidx = (iidx0, iidx1)
    urows = (urows0, urows1)
    vrows = (vrows0, vrows1)
    outv = (outv0, outv1)
    semi = (semi0, semi1)
    semg = (semg0, semg1)
    semo = (semo0, semo1)

    def cid_of(k):
        return wid + k * NW

    def valid(k):
        return cid_of(k) < NCHUNKS

    def start_idx(k, b):
        base = cid_of(k) * C
        pltpu.async_copy(ue_hbm.at[pl.ds(base, C)], uidx[b], semi[b])
        pltpu.async_copy(ie_hbm.at[pl.ds(base, C)], iidx[b], semi[b])

    def wait_idx(b):
        pltpu.make_async_copy(ue_hbm.at[pl.ds(0, C)], uidx[b], semi[b]).wait()
        pltpu.make_async_copy(ie_hbm.at[pl.ds(0, C)], iidx[b], semi[b]).wait()

    def start_gather(b):
        pltpu.async_copy(user_hbm.at[uidx[b]], urows[b], semg[b])
        pltpu.async_copy(item_hbm.at[iidx[b]], vrows[b], semg[b])

    def wait_gather(b):
        pltpu.make_async_copy(user_hbm.at[uidx[b]], urows[b], semg[b]).wait()
        pltpu.make_async_copy(item_hbm.at[iidx[b]], vrows[b], semg[b]).wait()

    def compute(b):
        def grp_body(g, _):
            out_vec = jnp.zeros((L,), jnp.float32)
            for e in range(L):
                row = g * L + e
                acc = urows[b][row, pl.ds(0, L)] * vrows[b][row, pl.ds(0, L)]
                for kk in range(1, D // L):
                    a = urows[b][row, pl.ds(kk * L, L)]
                    v = vrows[b][row, pl.ds(kk * L, L)]
                    acc = acc + a * v
                for p in perms:
                    acc = acc + _permute(acc, p)
                out_vec = jnp.where(lane == e, acc, out_vec)
            outv[b][pl.ds(g * L, L)] = out_vec
            return 0

        lax.fori_loop(0, C // L, grp_body, 0)

    def start_out(k, b):
        pltpu.async_copy(outv[b], out_hbm.at[pl.ds(cid_of(k) * C, C)],
                         semo[b])

    def wait_out(b):
        pltpu.make_async_copy(outv[b], out_hbm.at[pl.ds(0, C)],
                              semo[b]).wait()

    pltpu.sync_copy(ue_hbm.at[pl.ds(cid_of(0) * C, C)], uidx[0])
    pltpu.sync_copy(ie_hbm.at[pl.ds(cid_of(0) * C, C)], iidx[0])
    start_gather(0)
    start_idx(1, 1)

    def loop_body(i, _):
        for b in (0, 1):
            k = 2 * i + b

            @pl.when(valid(k + 1))
            def _():
                wait_idx(1 - b)
                start_gather(1 - b)

            @pl.when(valid(k))
            def _():
                wait_gather(b)

            @pl.when(valid(k + 2))
            def _():
                start_idx(k + 2, b)

            @pl.when(valid(k) & (k >= 2))
            def _():
                wait_out(b)

            @pl.when(valid(k))
            def _():
                compute(b)
                start_out(k, b)
        return 0

    lax.fori_loop(0, KPAIR, loop_body, 0)

    wait_out(0)
    wait_out(1)


def kernel(user_embedding, item_embedding, pos_edges, neg_edges):
    ue = jnp.concatenate([pos_edges[0], neg_edges[0]])
    ie = jnp.concatenate([pos_edges[1], neg_edges[1]])
    out = _edge_scores(user_embedding, item_embedding, ue, ie)
    return (out[:E, None], out[E:, None])


# speedup vs baseline: 2.7633x; 1.5537x over previous
import functools

import jax
import jax.numpy as jnp
from jax import lax
from jax.experimental import pallas as pl
from jax.experimental.pallas import tpu as pltpu
from jax.experimental.pallas import tpu_sc as plsc

D = 128
E = 160000
E2 = 2 * E
L = 16
NC, NS = 2, 16
NW = NC * NS
C = 128
NCHUNKS = E2 // C
K = -(-NCHUNKS // NW)
KPAIR = (K + 1) // 2
GRP = 8

_PERM_DNUMS = lax.GatherDimensionNumbers(
    offset_dims=(), collapsed_slice_dims=(0,), start_index_map=(0,))


def _permute(x, idx):
    return lax.gather(x, idx[:, None], _PERM_DNUMS, slice_sizes=(1,),
                      mode=lax.GatherScatterMode.PROMISE_IN_BOUNDS)


@functools.partial(
    pl.kernel,
    out_type=jax.ShapeDtypeStruct((E2,), jnp.float32),
    mesh=plsc.VectorSubcoreMesh(core_axis_name="c", subcore_axis_name="s"),
    scratch_types=[
        pltpu.VMEM((C,), jnp.int32), pltpu.VMEM((C,), jnp.int32),
        pltpu.VMEM((C,), jnp.int32), pltpu.VMEM((C,), jnp.int32),
        pltpu.VMEM((C, D), jnp.float32), pltpu.VMEM((C, D), jnp.float32),
        pltpu.VMEM((C, D), jnp.float32), pltpu.VMEM((C, D), jnp.float32),
        pltpu.VMEM((C + 8,), jnp.float32), pltpu.VMEM((C + 8,), jnp.float32),
        pltpu.SemaphoreType.DMA, pltpu.SemaphoreType.DMA,
        pltpu.SemaphoreType.DMA, pltpu.SemaphoreType.DMA,
        pltpu.SemaphoreType.DMA, pltpu.SemaphoreType.DMA,
    ],
)
def _edge_scores(user_hbm, item_hbm, ue_hbm, ie_hbm, out_hbm,
                 uidx0, uidx1, iidx0, iidx1,
                 urows0, urows1, vrows0, vrows1,
                 outv0, outv1,
                 semi0, semi1, semg0, semg1, semo0, semo1):
    wid = lax.axis_index("s") * NC + lax.axis_index("c")
    lane = lax.iota(jnp.int32, L)
    perms = [jnp.bitwise_xor(lane, s) for s in (8, 4, 2, 1)]

    uidx = (uidx0, uidx1)
    iidx = (iidx0, iidx1)
    urows = (urows0, urows1)
    vrows = (vrows0, vrows1)
    outv = (outv0, outv1)
    semi = (semi0, semi1)
    semg = (semg0, semg1)
    semo = (semo0, semo1)

    def cid_of(k):
        return wid + k * NW

    def valid(k):
        return cid_of(k) < NCHUNKS

    def start_idx(k, b):
        base = cid_of(k) * C
        pltpu.async_copy(ue_hbm.at[pl.ds(base, C)], uidx[b], semi[b])
        pltpu.async_copy(ie_hbm.at[pl.ds(base, C)], iidx[b], semi[b])

    def wait_idx(b):
        pltpu.make_async_copy(ue_hbm.at[pl.ds(0, C)], uidx[b], semi[b]).wait()
        pltpu.make_async_copy(ie_hbm.at[pl.ds(0, C)], iidx[b], semi[b]).wait()

    def start_gather(b):
        pltpu.async_copy(user_hbm.at[uidx[b]], urows[b], semg[b])
        pltpu.async_copy(item_hbm.at[iidx[b]], vrows[b], semg[b])

    def wait_gather(b):
        pltpu.make_async_copy(user_hbm.at[uidx[b]], urows[b], semg[b]).wait()
        pltpu.make_async_copy(item_hbm.at[iidx[b]], vrows[b], semg[b]).wait()

    def compute(b):
        def grp_body(g, _):
            out_vec = jnp.zeros((L,), jnp.float32)
            for e in range(GRP):
                row = g * GRP + e
                acc = urows[b][row, pl.ds(0, L)] * vrows[b][row, pl.ds(0, L)]
                for kk in range(1, D // L):
                    a = urows[b][row, pl.ds(kk * L, L)]
                    v = vrows[b][row, pl.ds(kk * L, L)]
                    acc = acc + a * v
                for p in perms:
                    acc = acc + _permute(acc, p)
                out_vec = jnp.where(lane == e, acc, out_vec)
            outv[b][pl.ds(g * GRP, L)] = out_vec
            return 0

        lax.fori_loop(0, C // GRP, grp_body, 0)

    def start_out(k, b):
        pltpu.async_copy(outv[b].at[pl.ds(0, C)],
                         out_hbm.at[pl.ds(cid_of(k) * C, C)], semo[b])

    def wait_out(b):
        pltpu.make_async_copy(outv[b].at[pl.ds(0, C)],
                              out_hbm.at[pl.ds(0, C)], semo[b]).wait()

    pltpu.sync_copy(ue_hbm.at[pl.ds(cid_of(0) * C, C)], uidx[0])
    pltpu.sync_copy(ie_hbm.at[pl.ds(cid_of(0) * C, C)], iidx[0])
    start_gather(0)
    start_idx(1, 1)

    def loop_body(i, _):
        for b in (0, 1):
            k = 2 * i + b

            @pl.when(valid(k + 1))
            def _():
                wait_idx(1 - b)
                start_gather(1 - b)

            @pl.when(valid(k))
            def _():
                wait_gather(b)

            @pl.when(valid(k + 2))
            def _():
                start_idx(k + 2, b)

            @pl.when(valid(k) & (k >= 2))
            def _():
                wait_out(b)

            @pl.when(valid(k))
            def _():
                compute(b)
                start_out(k, b)
        return 0

    lax.fori_loop(0, KPAIR, loop_body, 0)

    wait_out(0)
    wait_out(1)


def kernel(user_embedding, item_embedding, pos_edges, neg_edges):
    ue = jnp.concatenate([pos_edges[0], neg_edges[0]])
    ie = jnp.concatenate([pos_edges[1], neg_edges[1]])
    out = _edge_scores(user_embedding, item_embedding, ue, ie)
    return (out[:E, None], out[E:, None])


# speedup vs baseline: 2.8511x; 1.0318x over previous
import functools

import jax
import jax.numpy as jnp
from jax import lax
from jax.experimental import pallas as pl
from jax.experimental.pallas import tpu as pltpu
from jax.experimental.pallas import tpu_sc as plsc

D = 128
E = 160000
E2 = 2 * E
L = 16
NC, NS = 2, 16
NW = NC * NS
C = 200
NCHUNKS = E2 // C
K = -(-NCHUNKS // NW)
KPAIR = (K + 1) // 2
GRP = 8
SPLITS = ((0, 128), (128, C - 128))

_PERM_DNUMS = lax.GatherDimensionNumbers(
    offset_dims=(), collapsed_slice_dims=(0,), start_index_map=(0,))


def _permute(x, idx):
    return lax.gather(x, idx[:, None], _PERM_DNUMS, slice_sizes=(1,),
                      mode=lax.GatherScatterMode.PROMISE_IN_BOUNDS)


@functools.partial(
    pl.kernel,
    out_type=jax.ShapeDtypeStruct((E2,), jnp.float32),
    mesh=plsc.VectorSubcoreMesh(core_axis_name="c", subcore_axis_name="s"),
    scratch_types=[
        pltpu.VMEM((C,), jnp.int32), pltpu.VMEM((C,), jnp.int32),
        pltpu.VMEM((C,), jnp.int32), pltpu.VMEM((C,), jnp.int32),
        pltpu.VMEM((C, D), jnp.float32), pltpu.VMEM((C, D), jnp.float32),
        pltpu.VMEM((C, D), jnp.float32), pltpu.VMEM((C, D), jnp.float32),
        pltpu.VMEM((C + 8,), jnp.float32), pltpu.VMEM((C + 8,), jnp.float32),
        pltpu.SemaphoreType.DMA, pltpu.SemaphoreType.DMA,
        pltpu.SemaphoreType.DMA, pltpu.SemaphoreType.DMA,
        pltpu.SemaphoreType.DMA, pltpu.SemaphoreType.DMA,
    ],
)
def _edge_scores(user_hbm, item_hbm, ue_hbm, ie_hbm, out_hbm,
                 uidx0, uidx1, iidx0, iidx1,
                 urows0, urows1, vrows0, vrows1,
                 outv0, outv1,
                 semi0, semi1, semg0, semg1, semo0, semo1):
    wid = lax.axis_index("s") * NC + lax.axis_index("c")
    lane = lax.iota(jnp.int32, L)
    perms = [jnp.bitwise_xor(lane, s) for s in (8, 4, 2, 1)]

    uidx = (uidx0, uidx1)
    iidx = (iidx0, iidx1)
    urows = (urows0, urows1)
    vrows = (vrows0, vrows1)
    outv = (outv0, outv1)
    semi = (semi0, semi1)
    semg = (semg0, semg1)
    semo = (semo0, semo1)

    def cid_of(k):
        return wid + k * NW

    def valid(k):
        return cid_of(k) < NCHUNKS

    def start_idx(k, b):
        base = cid_of(k) * C
        pltpu.async_copy(ue_hbm.at[pl.ds(base, C)], uidx[b], semi[b])
        pltpu.async_copy(ie_hbm.at[pl.ds(base, C)], iidx[b], semi[b])

    def wait_idx(b):
        pltpu.make_async_copy(ue_hbm.at[pl.ds(0, C)], uidx[b], semi[b]).wait()
        pltpu.make_async_copy(ie_hbm.at[pl.ds(0, C)], iidx[b], semi[b]).wait()

    def start_gather(b):
        for off, cnt in SPLITS:
            pltpu.async_copy(user_hbm.at[uidx[b].at[pl.ds(off, cnt)]],
                             urows[b].at[pl.ds(off, cnt)], semg[b])
            pltpu.async_copy(item_hbm.at[iidx[b].at[pl.ds(off, cnt)]],
                             vrows[b].at[pl.ds(off, cnt)], semg[b])

    def wait_gather(b):
        for off, cnt in SPLITS:
            pltpu.make_async_copy(user_hbm.at[uidx[b].at[pl.ds(off, cnt)]],
                                  urows[b].at[pl.ds(off, cnt)],
                                  semg[b]).wait()
            pltpu.make_async_copy(item_hbm.at[iidx[b].at[pl.ds(off, cnt)]],
                                  vrows[b].at[pl.ds(off, cnt)],
                                  semg[b]).wait()

    def compute(b):
        def grp_body(g, _):
            out_vec = jnp.zeros((L,), jnp.float32)
            for e in range(GRP):
                row = g * GRP + e
                acc = urows[b][row, pl.ds(0, L)] * vrows[b][row, pl.ds(0, L)]
                for kk in range(1, D // L):
                    a = urows[b][row, pl.ds(kk * L, L)]
                    v = vrows[b][row, pl.ds(kk * L, L)]
                    acc = acc + a * v
                for p in perms:
                    acc = acc + _permute(acc, p)
                out_vec = jnp.where(lane == e, acc, out_vec)
            outv[b][pl.ds(g * GRP, L)] = out_vec
            return 0

        lax.fori_loop(0, C // GRP, grp_body, 0)

    def start_out(k, b):
        pltpu.async_copy(outv[b].at[pl.ds(0, C)],
                         out_hbm.at[pl.ds(cid_of(k) * C, C)], semo[b])

    def wait_out(b):
        pltpu.make_async_copy(outv[b].at[pl.ds(0, C)],
                              out_hbm.at[pl.ds(0, C)], semo[b]).wait()

    pltpu.sync_copy(ue_hbm.at[pl.ds(cid_of(0) * C, C)], uidx[0])
    pltpu.sync_copy(ie_hbm.at[pl.ds(cid_of(0) * C, C)], iidx[0])
    start_gather(0)
    start_idx(1, 1)

    def loop_body(i, _):
        for b in (0, 1):
            k = 2 * i + b

            @pl.when(valid(k + 1))
            def _():
                wait_idx(1 - b)
                start_gather(1 - b)

            @pl.when(valid(k))
            def _():
                wait_gather(b)

            @pl.when(valid(k + 2))
            def _():
                start_idx(k + 2, b)

            @pl.when(valid(k) & (k >= 2))
            def _():
                wait_out(b)

            @pl.when(valid(k))
            def _():
                compute(b)
                start_out(k, b)
        return 0

    lax.fori_loop(0, KPAIR, loop_body, 0)

    wait_out(0)
    wait_out(1)


def kernel(user_embedding, item_embedding, pos_edges, neg_edges):
    ue = jnp.concatenate([pos_edges[0], neg_edges[0]])
    ie = jnp.concatenate([pos_edges[1], neg_edges[1]])
    out = _edge_scores(user_embedding, item_embedding, ue, ie)
    return (out[:E, None], out[E:, None])
